# baseline (speedup 1.0000x reference)
import functools

import jax
import jax.numpy as jnp
from jax import lax
from jax.experimental import pallas as pl
from jax.experimental.pallas import tpu as pltpu
from jax.experimental.pallas import tpu_sc as plsc

F = 26
V = 100000
D = 128
B = 4096
NC = 2
NS = 16
NW = NC * NS
CB = B // NW
NSLOT = 4
SPLITS = (13, 13)

_PACK_ORDER = "identity"


def _w_perm():
  cols = jnp.arange(F * D)
  if _PACK_ORDER == "identity":
    return cols
  within = cols % 32
  first_half = (within % 2) if _PACK_ORDER == "a_low" else (1 - within % 2)
  return (cols - within) + first_half * 16 + within // 2


def _sc_gather(tab_flat, ids_chunk, f_base, fc):
  mesh = plsc.VectorSubcoreMesh(
      core_axis_name="c", subcore_axis_name="s", num_cores=NC, num_subcores=NS)

  @functools.partial(
      pl.kernel,
      out_type=jax.ShapeDtypeStruct((fc * B // 2, 2, D), jnp.bfloat16),
      mesh=mesh,
      scratch_types=[
          pltpu.VMEM((fc, CB), jnp.int32),
          pltpu.VMEM((NSLOT, CB, D), jnp.float32),
          pltpu.VMEM((NSLOT, CB // 2, 2, D), jnp.bfloat16),
          [pltpu.SemaphoreType.DMA] * NSLOT,
          [pltpu.SemaphoreType.DMA] * NSLOT,
      ],
  )
  def gather_k(tab_hbm, ids_hbm, out_hbm, idx_v, rows_v, bf_v, gsem, osem):
    w = lax.axis_index("s") * NC + lax.axis_index("c")
    col = w * CB
    pltpu.sync_copy(ids_hbm.at[:, pl.ds(col, CB)], idx_v)
    for f in range(fc):
      off = (f_base + f) * V
      if off == 0:
        continue
      for j in range(CB // 16):
        sl = pl.ds(j * 16, 16)
        idx_v[f, sl] = idx_v[f, sl] + off
    gcopies = [None] * fc
    ocopies = [None] * fc

    def start_out(f):
      slot = f % NSLOT
      gcopies[f].wait()

      def pack_row(r, _):
        for h in range(2):
          for cb in range(D // 16):
            sl = pl.ds(cb * 16, 16)
            bf_v[slot, r, h, sl] = (
                rows_v[slot, 2 * r + h, sl].astype(jnp.bfloat16))
        return 0

      lax.fori_loop(0, CB // 2, pack_row, 0)
      ocopies[f] = pltpu.async_copy(
          bf_v.at[slot],
          out_hbm.at[pl.ds(f * (B // 2) + w * (CB // 2), CB // 2)],
          osem[slot])

    for f in range(fc):
      slot = f % NSLOT
      if f >= NSLOT:
        ocopies[f - NSLOT].wait()
      gcopies[f] = pltpu.async_copy(
          tab_hbm.at[idx_v.at[f]], rows_v.at[slot], gsem[slot])
      if f >= NSLOT - 1:
        start_out(f - (NSLOT - 1))
    for f in range(max(fc - NSLOT + 1, 0), fc):
      start_out(f)
    for f in range(max(fc - NSLOT, 0), fc):
      ocopies[f].wait()

  return gather_k(tab_flat, ids_chunk)


def _mm_first_body(fc):
  def body(e_ref, w_ref, o_ref):
    f = pl.program_id(0)
    part = lax.dot_general(
        e_ref[0], w_ref[...],
        (((1,), (1,)), ((), ())),
        preferred_element_type=jnp.float32)

    @pl.when(f == 0)
    def _():
      o_ref[...] = part

    @pl.when(f > 0)
    def _():
      o_ref[...] = o_ref[...] + part

  return body


def _mm_next_body(fc, last):
  def body(e_ref, w_ref, b_ref, acc_ref, o_ref):
    f = pl.program_id(0)
    part = lax.dot_general(
        e_ref[0], w_ref[...],
        (((1,), (1,)), ((), ())),
        preferred_element_type=jnp.float32)

    @pl.when(f == 0)
    def _():
      o_ref[...] = acc_ref[...] + part

    @pl.when(f > 0)
    def _():
      o_ref[...] = o_ref[...] + part

    if last:
      @pl.when(f == fc - 1)
      def _():
        o_ref[...] = jnp.maximum(o_ref[...] + b_ref[...], 0.0)

  return body


def _tc_matmul_first(e3, w_full, fc, f_base):
  return pl.pallas_call(
      _mm_first_body(fc),
      grid=(fc,),
      in_specs=[
          pl.BlockSpec((1, B, D), lambda f: (f, 0, 0)),
          pl.BlockSpec((D, D), lambda f, fb=f_base: (0, fb + f)),
      ],
      out_specs=pl.BlockSpec((B, D), lambda f: (0, 0)),
      out_shape=jax.ShapeDtypeStruct((B, D), jnp.float32),
      compiler_params=pltpu.CompilerParams(
          dimension_semantics=("arbitrary",)),
  )(e3, w_full)


def _tc_matmul_next(e3, w_full, b2, acc, fc, f_base, last):
  return pl.pallas_call(
      _mm_next_body(fc, last),
      grid=(fc,),
      in_specs=[
          pl.BlockSpec((1, B, D), lambda f: (f, 0, 0)),
          pl.BlockSpec((D, D), lambda f, fb=f_base: (0, fb + f)),
          pl.BlockSpec((1, D), lambda f: (0, 0)),
          pl.BlockSpec((B, D), lambda f: (0, 0)),
      ],
      out_specs=pl.BlockSpec((B, D), lambda f: (0, 0)),
      out_shape=jax.ShapeDtypeStruct((B, D), jnp.float32),
      compiler_params=pltpu.CompilerParams(
          dimension_semantics=("arbitrary",)),
  )(e3, w_full, b2, acc)


def kernel(ids, tables, W, b):
  tab_flat = tables.reshape(F * V, D)
  b2 = b.reshape(1, D)
  w_bf = W[:, _w_perm()].astype(jnp.bfloat16)
  bases = []
  fb = 0
  for fc in SPLITS:
    bases.append(fb)
    fb += fc
  es = [_sc_gather(tab_flat, ids[f_base:f_base + fc], f_base, fc)
        for f_base, fc in zip(bases, SPLITS)]
  es = [e.reshape(fc, B, D) for e, fc in zip(es, SPLITS)]
  acc = _tc_matmul_first(es[0], w_bf, SPLITS[0], 0)
  for i in range(1, len(SPLITS)):
    fc, f_base = SPLITS[i], bases[i]
    acc = _tc_matmul_next(es[i], w_bf, b2, acc, fc, f_base,
                          last=(i == len(SPLITS) - 1))
  return acc


# speedup vs baseline: 2.8773x; 2.8773x over previous
import functools

import jax
import jax.numpy as jnp
from jax import lax
from jax.experimental import pallas as pl
from jax.experimental.pallas import tpu as pltpu
from jax.experimental.pallas import tpu_sc as plsc

F = 26
V = 100000
D = 128
B = 4096
NC = 2
NS = 16
NW = NC * NS
CB = B // NW
NSLOT = 6
SPLITS = (13, 13)

_PACK_ORDER = "identity"


def _w_perm():
  cols = jnp.arange(F * D)
  if _PACK_ORDER == "identity":
    return cols
  within = cols % 32
  first_half = (within % 2) if _PACK_ORDER == "a_low" else (1 - within % 2)
  return (cols - within) + first_half * 16 + within // 2


def _sc_gather(tab_flat, ids_chunk, f_base, fc):
  mesh = plsc.VectorSubcoreMesh(
      core_axis_name="c", subcore_axis_name="s", num_cores=NC, num_subcores=NS)

  @functools.partial(
      pl.kernel,
      out_type=jax.ShapeDtypeStruct((fc * B, D), jnp.float32),
      mesh=mesh,
      scratch_types=[
          pltpu.VMEM((fc, CB), jnp.int32),
          pltpu.VMEM((NSLOT, CB, D), jnp.float32),
          [pltpu.SemaphoreType.DMA] * NSLOT,
          [pltpu.SemaphoreType.DMA] * NSLOT,
      ],
  )
  def gather_k(tab_hbm, ids_hbm, out_hbm, idx_v, rows_v, gsem, osem):
    w = lax.axis_index("s") * NC + lax.axis_index("c")
    col = w * CB
    pltpu.sync_copy(ids_hbm.at[:, pl.ds(col, CB)], idx_v)
    for f in range(fc):
      off = (f_base + f) * V
      if off == 0:
        continue
      for j in range(CB // 16):
        sl = pl.ds(j * 16, 16)
        idx_v[f, sl] = idx_v[f, sl] + off
    gcopies = [None] * fc
    ocopies = [None] * fc

    def start_out(f):
      slot = f % NSLOT
      gcopies[f].wait()
      ocopies[f] = pltpu.async_copy(
          rows_v.at[slot],
          out_hbm.at[pl.ds(f * B + col, CB)],
          osem[slot])

    for f in range(fc):
      slot = f % NSLOT
      if f >= NSLOT:
        ocopies[f - NSLOT].wait()
      gcopies[f] = pltpu.async_copy(
          tab_hbm.at[idx_v.at[f]], rows_v.at[slot], gsem[slot])
      if f >= NSLOT - 1:
        start_out(f - (NSLOT - 1))
    for f in range(max(fc - NSLOT + 1, 0), fc):
      start_out(f)
    for f in range(max(fc - NSLOT, 0), fc):
      ocopies[f].wait()

  return gather_k(tab_flat, ids_chunk)


def _mm_first_body(fc):
  def body(e_ref, w_ref, o_ref):
    f = pl.program_id(0)
    part = lax.dot_general(
        e_ref[0], w_ref[...],
        (((1,), (1,)), ((), ())),
        preferred_element_type=jnp.float32)

    @pl.when(f == 0)
    def _():
      o_ref[...] = part

    @pl.when(f > 0)
    def _():
      o_ref[...] = o_ref[...] + part

  return body


def _mm_next_body(fc, last):
  def body(e_ref, w_ref, b_ref, acc_ref, o_ref):
    f = pl.program_id(0)
    part = lax.dot_general(
        e_ref[0], w_ref[...],
        (((1,), (1,)), ((), ())),
        preferred_element_type=jnp.float32)

    @pl.when(f == 0)
    def _():
      o_ref[...] = acc_ref[...] + part

    @pl.when(f > 0)
    def _():
      o_ref[...] = o_ref[...] + part

    if last:
      @pl.when(f == fc - 1)
      def _():
        o_ref[...] = jnp.maximum(o_ref[...] + b_ref[...], 0.0)

  return body


def _tc_matmul_first(e3, w_full, fc, f_base):
  return pl.pallas_call(
      _mm_first_body(fc),
      grid=(fc,),
      in_specs=[
          pl.BlockSpec((1, B, D), lambda f: (f, 0, 0)),
          pl.BlockSpec((D, D), lambda f, fb=f_base: (0, fb + f)),
      ],
      out_specs=pl.BlockSpec((B, D), lambda f: (0, 0)),
      out_shape=jax.ShapeDtypeStruct((B, D), jnp.float32),
      compiler_params=pltpu.CompilerParams(
          dimension_semantics=("arbitrary",)),
  )(e3, w_full)


def _tc_matmul_next(e3, w_full, b2, acc, fc, f_base, last):
  return pl.pallas_call(
      _mm_next_body(fc, last),
      grid=(fc,),
      in_specs=[
          pl.BlockSpec((1, B, D), lambda f: (f, 0, 0)),
          pl.BlockSpec((D, D), lambda f, fb=f_base: (0, fb + f)),
          pl.BlockSpec((1, D), lambda f: (0, 0)),
          pl.BlockSpec((B, D), lambda f: (0, 0)),
      ],
      out_specs=pl.BlockSpec((B, D), lambda f: (0, 0)),
      out_shape=jax.ShapeDtypeStruct((B, D), jnp.float32),
      compiler_params=pltpu.CompilerParams(
          dimension_semantics=("arbitrary",)),
  )(e3, w_full, b2, acc)


def kernel(ids, tables, W, b):
  tab_flat = tables.reshape(F * V, D)
  b2 = b.reshape(1, D)
  w_bf = W
  bases = []
  fb = 0
  for fc in SPLITS:
    bases.append(fb)
    fb += fc
  es = [_sc_gather(tab_flat, ids[f_base:f_base + fc], f_base, fc)
        for f_base, fc in zip(bases, SPLITS)]
  es = [e.reshape(fc, B, D) for e, fc in zip(es, SPLITS)]
  acc = _tc_matmul_first(es[0], w_bf, SPLITS[0], 0)
  for i in range(1, len(SPLITS)):
    fc, f_base = SPLITS[i], bases[i]
    acc = _tc_matmul_next(es[i], w_bf, b2, acc, fc, f_base,
                          last=(i == len(SPLITS) - 1))
  return acc
